# Initial kernel scaffold; baseline (speedup 1.0000x reference)
import functools

import jax
import jax.numpy as jnp
from jax import lax
from jax.experimental import pallas as pl
from jax.experimental.pallas import tpu as pltpu
from jax.experimental.pallas import tpu_sc as plsc

NC = 2
NS = 16
CHUNK = 128

N_NODES = 10000
D = 128
N_PAD = 10240
ROWS_PER_TILE = N_PAD // NS


def _sc_aggregate(n_chunks):
  mesh = plsc.VectorSubcoreMesh(core_axis_name="c", subcore_axis_name="s")

  def body(x_hbm, seg_hbm, gid_hbm, zsum_hbm, zcnt_hbm, ones_hbm,
           out_sum, out_cnt,
           seg_v, gid_v, rows_v, ones_v, acc, cnt, sem):
    cid = lax.axis_index("c")
    sid = lax.axis_index("s")
    row_lo = sid * ROWS_PER_TILE

    pltpu.sync_copy(zsum_hbm.at[pl.ds(row_lo, ROWS_PER_TILE)],
                    acc.at[pl.ds(row_lo, ROWS_PER_TILE)])
    pltpu.sync_copy(zcnt_hbm.at[pl.ds(row_lo, ROWS_PER_TILE)],
                    cnt.at[pl.ds(row_lo, ROWS_PER_TILE)])

    pltpu.sync_copy(seg_hbm.at[cid, sid], seg_v)
    pltpu.sync_copy(gid_hbm.at[cid, sid], gid_v)
    pltpu.sync_copy(ones_hbm, ones_v)
    plsc.subcore_barrier()

    def step(j, carry):
      pltpu.async_copy(x_hbm.at[gid_v.at[j]], rows_v, sem).wait()
      pltpu.sync_copy(rows_v, acc.at[seg_v.at[j]], add=True)
      pltpu.sync_copy(ones_v, cnt.at[seg_v.at[j]], add=True)
      return carry

    lax.fori_loop(0, n_chunks, step, 0)
    plsc.subcore_barrier()

    pltpu.sync_copy(acc.at[pl.ds(row_lo, ROWS_PER_TILE)],
                    out_sum.at[cid, pl.ds(row_lo, ROWS_PER_TILE)])
    pltpu.sync_copy(cnt.at[pl.ds(row_lo, ROWS_PER_TILE)],
                    out_cnt.at[cid, pl.ds(row_lo, ROWS_PER_TILE)])

  return pl.kernel(
      body,
      out_type=(
          jax.ShapeDtypeStruct((NC, N_PAD, D), jnp.float32),
          jax.ShapeDtypeStruct((NC, N_PAD, 16), jnp.float32),
      ),
      mesh=mesh,
      scratch_types=[
          pltpu.VMEM((n_chunks, CHUNK), jnp.int32),
          pltpu.VMEM((n_chunks, CHUNK), jnp.int32),
          pltpu.VMEM((CHUNK, D), jnp.float32),
          pltpu.VMEM((CHUNK, 16), jnp.float32),
          pltpu.VMEM_SHARED((N_PAD, D), jnp.float32),
          pltpu.VMEM_SHARED((N_PAD, 16), jnp.float32),
          pltpu.SemaphoreType.DMA,
      ],
  )


def _tc_body(x_ref, p0_ref, p1_ref, c0_ref, c1_ref, ws_ref, wn_ref, o_ref):
  x = x_ref[...]
  sums = p0_ref[0] + p1_ref[0]
  counts = c0_ref[0][:, 0:1] + c1_ref[0][:, 0:1]
  mean = sums / jnp.maximum(counts, 1.0)
  h = jnp.dot(mean, wn_ref[...], preferred_element_type=jnp.float32)
  h = jnp.where(h >= 0, h, 0.2 * h)
  s = jnp.dot(x, ws_ref[...], preferred_element_type=jnp.float32)
  u = s + h
  u = jnp.where(u >= 0, u, 0.2 * u)
  nrm = jnp.sqrt(jnp.sum(u * u, axis=1, keepdims=True))
  o_ref[...] = u / jnp.maximum(nrm, 1e-12)


def kernel(node_fts, edge_fts, edges, W_self, W_neigh):
  del edge_fts
  n = node_fts.shape[0]
  e2 = 2 * edges.shape[1]

  n_chunks = -(-e2 // (NC * NS * CHUNK))
  e_pad = n_chunks * NC * NS * CHUNK

  seg = jnp.concatenate([edges[0], edges[1]])
  gid = jnp.concatenate([edges[1], edges[0]])
  pad = e_pad - e2
  seg = jnp.concatenate([seg, jnp.full((pad,), N_PAD - 8, jnp.int32)])
  gid = jnp.concatenate([gid, jnp.zeros((pad,), jnp.int32)])
  seg = seg.reshape(NC, NS, n_chunks, CHUNK)
  gid = gid.reshape(NC, NS, n_chunks, CHUNK)

  zsum = jnp.zeros((N_PAD, D), jnp.float32)
  zcnt = jnp.zeros((N_PAD, 16), jnp.float32)
  ones = jnp.ones((CHUNK, 16), jnp.float32)

  part_sum, part_cnt = _sc_aggregate(n_chunks)(
      node_fts, seg, gid, zsum, zcnt, ones)

  bn = 1000
  grid = n // bn
  out = pl.pallas_call(
      _tc_body,
      grid=(grid,),
      in_specs=[
          pl.BlockSpec((bn, D), lambda i: (i, 0)),
          pl.BlockSpec((1, bn, D), lambda i: (0, i, 0)),
          pl.BlockSpec((1, bn, D), lambda i: (1, i, 0)),
          pl.BlockSpec((1, bn, 16), lambda i: (0, i, 0)),
          pl.BlockSpec((1, bn, 16), lambda i: (1, i, 0)),
          pl.BlockSpec((D, D), lambda i: (0, 0)),
          pl.BlockSpec((D, D), lambda i: (0, 0)),
      ],
      out_specs=pl.BlockSpec((bn, D), lambda i: (i, 0)),
      out_shape=jax.ShapeDtypeStruct((n, D), jnp.float32),
  )(node_fts, part_sum, part_sum, part_cnt, part_cnt,
    W_self.T, W_neigh.T)
  return out


# speedup vs baseline: 5.6874x; 5.6874x over previous
import jax
import jax.numpy as jnp
from jax import lax
from jax.experimental import pallas as pl
from jax.experimental.pallas import tpu as pltpu
from jax.experimental.pallas import tpu_sc as plsc

NC = 2
NS = 16
CHUNK = 128
K_GROUP = 16

D = 128
N_PAD = 10240
ROWS_PER_TILE = N_PAD // NS


def _sc_aggregate(n_groups):
  mesh = plsc.VectorSubcoreMesh(core_axis_name="c", subcore_axis_name="s")

  def body(x_hbm, seg_hbm, gid_hbm, zrow_hbm, ones_hbm,
           out_sum, out_cnt,
           seg_v, gid_v, rows_v, acc, sem):
    cid = lax.axis_index("c")
    sid = lax.axis_index("s")
    row_lo = sid * ROWS_PER_TILE

    def zero_acc():
      pltpu.sync_copy(zrow_hbm, rows_v)
      for i in range(ROWS_PER_TILE // CHUNK):
        pltpu.async_copy(rows_v, acc.at[pl.ds(row_lo + i * CHUNK, CHUNK)],
                         sem).wait()

    def writeback(out_hbm):
      for i in range(ROWS_PER_TILE // CHUNK):
        lo = row_lo + i * CHUNK
        pltpu.async_copy(acc.at[pl.ds(lo, CHUNK)], rows_v, sem).wait()
        pltpu.sync_copy(rows_v, out_hbm.at[cid, pl.ds(lo, CHUNK)])

    zero_acc()
    plsc.subcore_barrier()

    def group_sum(g, carry):
      pltpu.sync_copy(seg_hbm.at[cid, sid, pl.ds(g * K_GROUP, K_GROUP)],
                      seg_v)
      pltpu.sync_copy(gid_hbm.at[cid, sid, pl.ds(g * K_GROUP, K_GROUP)],
                      gid_v)

      def step(j, c2):
        pltpu.async_copy(x_hbm.at[gid_v.at[j]], rows_v, sem).wait()
        pltpu.async_copy(rows_v, acc.at[seg_v.at[j]], sem, add=True).wait()
        return c2

      lax.fori_loop(0, K_GROUP, step, 0)
      return carry

    lax.fori_loop(0, n_groups, group_sum, 0)
    plsc.subcore_barrier()
    writeback(out_sum)

    zero_acc()
    plsc.subcore_barrier()
    pltpu.sync_copy(ones_hbm, rows_v)

    def group_cnt(g, carry):
      pltpu.sync_copy(seg_hbm.at[cid, sid, pl.ds(g * K_GROUP, K_GROUP)],
                      seg_v)

      def step(j, c2):
        pltpu.async_copy(rows_v, acc.at[seg_v.at[j]], sem, add=True).wait()
        return c2

      lax.fori_loop(0, K_GROUP, step, 0)
      return carry

    lax.fori_loop(0, n_groups, group_cnt, 0)
    plsc.subcore_barrier()
    pltpu.sync_copy(zrow_hbm, rows_v)
    writeback(out_cnt)

  return pl.kernel(
      body,
      out_type=(
          jax.ShapeDtypeStruct((NC, N_PAD, D), jnp.float32),
          jax.ShapeDtypeStruct((NC, N_PAD, D), jnp.float32),
      ),
      mesh=mesh,
      scratch_types=[
          pltpu.VMEM((K_GROUP, CHUNK), jnp.int32),
          pltpu.VMEM((K_GROUP, CHUNK), jnp.int32),
          pltpu.VMEM((CHUNK, D), jnp.float32),
          pltpu.VMEM_SHARED((N_PAD, D), jnp.float32),
          pltpu.SemaphoreType.DMA,
      ],
  )


def _tc_body(x_ref, p0_ref, p1_ref, c0_ref, c1_ref, ws_ref, wn_ref, o_ref):
  x = x_ref[...]
  sums = p0_ref[0] + p1_ref[0]
  counts = c0_ref[0][:, 0:1] + c1_ref[0][:, 0:1]
  mean = sums / jnp.maximum(counts, 1.0)
  h = jnp.dot(mean, wn_ref[...], preferred_element_type=jnp.float32)
  h = jnp.where(h >= 0, h, 0.2 * h)
  s = jnp.dot(x, ws_ref[...], preferred_element_type=jnp.float32)
  u = s + h
  u = jnp.where(u >= 0, u, 0.2 * u)
  nrm = jnp.sqrt(jnp.sum(u * u, axis=1, keepdims=True))
  o_ref[...] = u / jnp.maximum(nrm, 1e-12)


def kernel(node_fts, edge_fts, edges, W_self, W_neigh):
  del edge_fts
  n = node_fts.shape[0]
  e2 = 2 * edges.shape[1]

  per_group = NC * NS * CHUNK * K_GROUP
  n_groups = -(-e2 // per_group)
  n_chunks = n_groups * K_GROUP
  e_pad = n_groups * per_group

  seg = jnp.concatenate([edges[0], edges[1]])
  gid = jnp.concatenate([edges[1], edges[0]])
  pad = e_pad - e2
  seg = jnp.concatenate([seg, jnp.full((pad,), N_PAD - 8, jnp.int32)])
  gid = jnp.concatenate([gid, jnp.zeros((pad,), jnp.int32)])
  seg = seg.reshape(NC, NS, n_chunks, CHUNK)
  gid = gid.reshape(NC, NS, n_chunks, CHUNK)

  zrow = jnp.zeros((CHUNK, D), jnp.float32)
  ones = jnp.ones((CHUNK, D), jnp.float32)

  part_sum, part_cnt = _sc_aggregate(n_groups)(
      node_fts, seg, gid, zrow, ones)

  bn = 1000
  grid = n // bn
  out = pl.pallas_call(
      _tc_body,
      grid=(grid,),
      in_specs=[
          pl.BlockSpec((bn, D), lambda i: (i, 0)),
          pl.BlockSpec((1, bn, D), lambda i: (0, i, 0)),
          pl.BlockSpec((1, bn, D), lambda i: (1, i, 0)),
          pl.BlockSpec((1, bn, D), lambda i: (0, i, 0)),
          pl.BlockSpec((1, bn, D), lambda i: (1, i, 0)),
          pl.BlockSpec((D, D), lambda i: (0, 0)),
          pl.BlockSpec((D, D), lambda i: (0, 0)),
      ],
      out_specs=pl.BlockSpec((bn, D), lambda i: (i, 0)),
      out_shape=jax.ShapeDtypeStruct((n, D), jnp.float32),
  )(node_fts, part_sum, part_sum, part_cnt, part_cnt,
    W_self.T, W_neigh.T)
  return out


# speedup vs baseline: 6.2364x; 1.0965x over previous
import jax
import jax.numpy as jnp
from jax import lax
from jax.experimental import pallas as pl
from jax.experimental.pallas import tpu as pltpu
from jax.experimental.pallas import tpu_sc as plsc

NC = 2
NS = 16
CHUNK = 128
K_GROUP = 16

D = 128
N_PAD = 10240
ROWS_PER_TILE = N_PAD // NS
ZCHUNKS = ROWS_PER_TILE // CHUNK


def _sc_aggregate(n_groups):
  mesh = plsc.VectorSubcoreMesh(core_axis_name="c", subcore_axis_name="s")

  def body(x_hbm, seg_hbm, gid_hbm, zrow_hbm, ones_hbm,
           out_sum, out_cnt,
           seg_v, gid_v, rows_a, rows_b, sem_g, sem_s, acc):
    cid = lax.axis_index("c")
    sid = lax.axis_index("s")
    row_lo = sid * ROWS_PER_TILE

    def zero_acc():
      pltpu.sync_copy(zrow_hbm, rows_a)
      for i in range(ZCHUNKS):
        pltpu.async_copy(rows_a, acc.at[pl.ds(row_lo + i * CHUNK, CHUNK)],
                         sem_s)
      for i in range(ZCHUNKS):
        pltpu.make_async_copy(
            rows_a, acc.at[pl.ds(row_lo, CHUNK)], sem_s).wait()

    def writeback(out_hbm):
      pltpu.async_copy(acc.at[pl.ds(row_lo, CHUNK)], rows_a, sem_g)
      for i in range(ZCHUNKS):
        buf = rows_a if i % 2 == 0 else rows_b
        nxt = rows_b if i % 2 == 0 else rows_a
        pltpu.make_async_copy(
            acc.at[pl.ds(row_lo, CHUNK)], buf, sem_g).wait()
        if i + 1 < ZCHUNKS:
          pltpu.async_copy(
              acc.at[pl.ds(row_lo + (i + 1) * CHUNK, CHUNK)], nxt, sem_g)
        pltpu.sync_copy(buf, out_hbm.at[cid, pl.ds(row_lo + i * CHUNK,
                                                   CHUNK)])

    zero_acc()
    plsc.subcore_barrier()

    def group_sum(g, carry):
      pltpu.sync_copy(seg_hbm.at[cid, sid, pl.ds(g * K_GROUP, K_GROUP)],
                      seg_v)
      pltpu.sync_copy(gid_hbm.at[cid, sid, pl.ds(g * K_GROUP, K_GROUP)],
                      gid_v)
      pltpu.async_copy(x_hbm.at[gid_v.at[0]], rows_a, sem_g)
      for k in range(K_GROUP):
        buf = rows_a if k % 2 == 0 else rows_b
        nxt = rows_b if k % 2 == 0 else rows_a
        pltpu.make_async_copy(x_hbm.at[gid_v.at[k]], buf, sem_g).wait()
        pltpu.async_copy(buf, acc.at[seg_v.at[k]], sem_s, add=True)
        if k >= 1:
          pltpu.make_async_copy(nxt, acc.at[seg_v.at[k]], sem_s).wait()
        if k + 1 < K_GROUP:
          pltpu.async_copy(x_hbm.at[gid_v.at[k + 1]], nxt, sem_g)
      pltpu.make_async_copy(rows_a, acc.at[seg_v.at[0]], sem_s).wait()
      return carry

    lax.fori_loop(0, n_groups, group_sum, 0)
    plsc.subcore_barrier()
    writeback(out_sum)

    zero_acc()
    plsc.subcore_barrier()
    pltpu.sync_copy(ones_hbm, rows_a)

    def group_cnt(g, carry):
      pltpu.sync_copy(seg_hbm.at[cid, sid, pl.ds(g * K_GROUP, K_GROUP)],
                      seg_v)
      for k in range(K_GROUP):
        pltpu.async_copy(rows_a, acc.at[seg_v.at[k]], sem_s, add=True)
      for k in range(K_GROUP):
        pltpu.make_async_copy(rows_a, acc.at[seg_v.at[0]], sem_s).wait()
      return carry

    lax.fori_loop(0, n_groups, group_cnt, 0)
    plsc.subcore_barrier()
    writeback(out_cnt)

  return pl.kernel(
      body,
      out_type=(
          jax.ShapeDtypeStruct((NC, N_PAD, D), jnp.float32),
          jax.ShapeDtypeStruct((NC, N_PAD, D), jnp.float32),
      ),
      mesh=mesh,
      scratch_types=[
          pltpu.VMEM((K_GROUP, CHUNK), jnp.int32),
          pltpu.VMEM((K_GROUP, CHUNK), jnp.int32),
          pltpu.VMEM((CHUNK, D), jnp.float32),
          pltpu.VMEM((CHUNK, D), jnp.float32),
          pltpu.SemaphoreType.DMA,
          pltpu.SemaphoreType.DMA,
          pltpu.VMEM_SHARED((N_PAD, D), jnp.float32),
      ],
  )


def _tc_body(x_ref, p0_ref, p1_ref, c0_ref, c1_ref, ws_ref, wn_ref, o_ref):
  x = x_ref[...]
  sums = p0_ref[0] + p1_ref[0]
  counts = c0_ref[0][:, 0:1] + c1_ref[0][:, 0:1]
  mean = sums / jnp.maximum(counts, 1.0)
  h = jnp.dot(mean, wn_ref[...], preferred_element_type=jnp.float32)
  h = jnp.where(h >= 0, h, 0.2 * h)
  s = jnp.dot(x, ws_ref[...], preferred_element_type=jnp.float32)
  u = s + h
  u = jnp.where(u >= 0, u, 0.2 * u)
  nrm = jnp.sqrt(jnp.sum(u * u, axis=1, keepdims=True))
  o_ref[...] = u / jnp.maximum(nrm, 1e-12)


def kernel(node_fts, edge_fts, edges, W_self, W_neigh):
  del edge_fts
  n = node_fts.shape[0]
  e2 = 2 * edges.shape[1]

  per_group = NC * NS * CHUNK * K_GROUP
  n_groups = -(-e2 // per_group)
  n_chunks = n_groups * K_GROUP
  e_pad = n_groups * per_group

  seg = jnp.concatenate([edges[0], edges[1]])
  gid = jnp.concatenate([edges[1], edges[0]])
  pad = e_pad - e2
  seg = jnp.concatenate([seg, jnp.full((pad,), N_PAD - 8, jnp.int32)])
  gid = jnp.concatenate([gid, jnp.zeros((pad,), jnp.int32)])
  seg = seg.reshape(NC, NS, n_chunks, CHUNK)
  gid = gid.reshape(NC, NS, n_chunks, CHUNK)

  zrow = jnp.zeros((CHUNK, D), jnp.float32)
  ones = jnp.ones((CHUNK, D), jnp.float32)

  part_sum, part_cnt = _sc_aggregate(n_groups)(
      node_fts, seg, gid, zrow, ones)

  bn = 1000
  grid = n // bn
  out = pl.pallas_call(
      _tc_body,
      grid=(grid,),
      in_specs=[
          pl.BlockSpec((bn, D), lambda i: (i, 0)),
          pl.BlockSpec((1, bn, D), lambda i: (0, i, 0)),
          pl.BlockSpec((1, bn, D), lambda i: (1, i, 0)),
          pl.BlockSpec((1, bn, D), lambda i: (0, i, 0)),
          pl.BlockSpec((1, bn, D), lambda i: (1, i, 0)),
          pl.BlockSpec((D, D), lambda i: (0, 0)),
          pl.BlockSpec((D, D), lambda i: (0, 0)),
      ],
      out_specs=pl.BlockSpec((bn, D), lambda i: (i, 0)),
      out_shape=jax.ShapeDtypeStruct((n, D), jnp.float32),
  )(node_fts, part_sum, part_sum, part_cnt, part_cnt,
    W_self.T, W_neigh.T)
  return out
